# baseline (speedup 1.0000x reference)
import jax
import jax.numpy as jnp
from jax.experimental import pallas as pl
from jax.experimental.pallas import tpu as pltpu

_VOCAB = 1000000
_DIM = 64
_CTX = 200
_NB = 125
_BP = 4000
_PACKED = _VOCAB // 2
_CH = _BP // 4
_OR = 8 * _NB


def _cbow_kernel(row2_ref, emb_hbm, mask_ref, w_ref, b_ref, out_ref,
                 m2_ref, rows_ref, stat_ref, sem):
    i = pl.program_id(0)

    @pl.when(i == 0)
    def _():
        def issue(j, c):
            pltpu.make_async_copy(
                emb_hbm.at[pl.ds(row2_ref[j], 1), :],
                rows_ref.at[pl.ds(j, 1), :], sem).start()
            return c

        jax.lax.fori_loop(0, _CTX, issue, 0)

        def wait(j, c):
            pltpu.make_async_copy(
                emb_hbm.at[pl.ds(row2_ref[j], 1), :],
                rows_ref.at[pl.ds(j, 1), :], sem).wait()
            return c

        jax.lax.fori_loop(0, _CTX, wait, 0)
        msum = jnp.sum(rows_ref[...] * mask_ref[...], axis=0, keepdims=True)
        m = (msum[:, :_DIM] + msum[:, _DIM:]) * (1.0 / _CTX)
        zero = jnp.zeros_like(m)
        m2_ref[...] = jnp.concatenate(
            [jnp.concatenate([m, zero], axis=1),
             jnp.concatenate([zero, m], axis=1)], axis=0).astype(jnp.bfloat16)
        stat_ref[0] = -jnp.inf
        stat_ref[1] = 0.0

    row = pl.multiple_of(8 * i, 8)
    s8 = jnp.concatenate([
        jax.lax.dot_general(
            m2_ref[...],
            w_ref[_CH * j:_CH * (j + 1), :].astype(jnp.bfloat16),
            (((1,), (1,)), ((), ())),
            preferred_element_type=jnp.float32,
        ) for j in range(4)
    ], axis=0) + b_ref[pl.ds(row, 8), :]
    out_ref[pl.ds(row, 8), :] = s8

    old_max = stat_ref[0]
    new_max = jnp.maximum(old_max, jnp.max(s8))
    stat_ref[1] = stat_ref[1] * jnp.exp(old_max - new_max) + jnp.sum(
        jnp.exp(s8 - new_max))
    stat_ref[0] = new_max

    @pl.when(i == _NB - 1)
    def _():
        lse = stat_ref[0] + jnp.log(stat_ref[1])
        out_ref[...] = out_ref[...] - lse


@jax.jit
def kernel(inputs, emb_table, W, b):
    idx = inputs.astype(jnp.int32)
    row2 = idx // 2
    odd = (idx % 2).astype(jnp.float32)
    lane = jnp.arange(2 * _DIM, dtype=jnp.int32)[None, :]
    mask = jnp.where(lane < _DIM, 1.0 - odd[:, None], odd[:, None])

    b_ro = (b.reshape(_NB, 4, _CH, 2).transpose(0, 1, 3, 2)
            .reshape(_OR, _CH))

    out = pl.pallas_call(
        _cbow_kernel,
        grid_spec=pltpu.PrefetchScalarGridSpec(
            num_scalar_prefetch=1,
            grid=(_NB,),
            in_specs=[
                pl.BlockSpec(memory_space=pl.ANY),
                pl.BlockSpec((_CTX, 2 * _DIM), lambda i, row2_ref: (0, 0)),
                pl.BlockSpec((_BP, 2 * _DIM), lambda i, row2_ref: (i, 0)),
                pl.BlockSpec((_OR, _CH), lambda i, row2_ref: (0, 0)),
            ],
            out_specs=pl.BlockSpec((_OR, _CH), lambda i, row2_ref: (0, 0)),
            scratch_shapes=[
                pltpu.VMEM((2, 2 * _DIM), jnp.bfloat16),
                pltpu.VMEM((_CTX, 2 * _DIM), jnp.float32),
                pltpu.SMEM((2,), jnp.float32),
                pltpu.SemaphoreType.DMA,
            ],
        ),
        out_shape=jax.ShapeDtypeStruct((_OR, _CH), jnp.float32),
    )(row2, emb_table.reshape(_PACKED, 2 * _DIM), mask,
      W.reshape(_PACKED, 2 * _DIM), b_ro)

    return (out.reshape(_NB, 4, 2, _CH).transpose(0, 1, 3, 2)
            .reshape(1, _VOCAB))


# speedup vs baseline: 8.8239x; 8.8239x over previous
import jax
import jax.numpy as jnp
from jax.experimental import pallas as pl
from jax.experimental.pallas import tpu as pltpu

_VOCAB = 1000000
_DIM = 64
_CTX = 200
_BN = 8192
_NB = -(-_VOCAB // _BN)


def _main_kernel(idx_ref, embt_hbm, mask_ref, wt_ref, b_ref, out_ref, lse_ref,
                 m_ref, cols_ref, stat_ref, sem):
    i = pl.program_id(0)

    @pl.when(i == 0)
    def _():
        def issue(j, c):
            base = pl.multiple_of(idx_ref[j] // 128 * 128, 128)
            pltpu.make_async_copy(
                embt_hbm.at[:, pl.ds(base, 128)],
                cols_ref.at[:, pl.ds(pl.multiple_of(128 * j, 128), 128)],
                sem).start()
            return c

        jax.lax.fori_loop(0, _CTX, issue, 0)

        def wait(j, c):
            base = pl.multiple_of(idx_ref[j] // 128 * 128, 128)
            pltpu.make_async_copy(
                embt_hbm.at[:, pl.ds(base, 128)],
                cols_ref.at[:, pl.ds(pl.multiple_of(128 * j, 128), 128)],
                sem).wait()
            return c

        jax.lax.fori_loop(0, _CTX, wait, 0)
        m_ref[...] = jnp.sum(cols_ref[...] * mask_ref[...], axis=1,
                             keepdims=True) * (1.0 / _CTX)
        stat_ref[0] = -jnp.inf
        stat_ref[1] = 0.0

    s = jax.lax.dot_general(
        m_ref[...], wt_ref[...], (((0,), (0,)), ((), ())),
        preferred_element_type=jnp.float32,
    ) + b_ref[...]
    out_ref[...] = s

    col = _BN * i + jax.lax.broadcasted_iota(jnp.int32, (1, _BN), 1)
    sm = jnp.where(col < _VOCAB, s, -jnp.inf)
    old_max = stat_ref[0]
    new_max = jnp.maximum(old_max, jnp.max(sm))
    stat_ref[1] = stat_ref[1] * jnp.exp(old_max - new_max) + jnp.sum(
        jnp.where(col < _VOCAB, jnp.exp(sm - new_max), 0.0))
    stat_ref[0] = new_max

    @pl.when(i == _NB - 1)
    def _():
        lse_ref[0, 0] = stat_ref[0] + jnp.log(stat_ref[1])


def _sub_kernel(x_ref, lse_ref, o_ref):
    o_ref[...] = x_ref[...] - lse_ref[0, 0]


@jax.jit
def kernel(inputs, emb_table, W, b):
    idx = inputs.astype(jnp.int32)
    onehot = (idx[:, None] % 128 ==
              jnp.arange(128, dtype=jnp.int32)[None, :]).astype(jnp.float32)
    mask = onehot.reshape(1, _CTX * 128)

    logits, lse = pl.pallas_call(
        _main_kernel,
        grid_spec=pltpu.PrefetchScalarGridSpec(
            num_scalar_prefetch=1,
            grid=(_NB,),
            in_specs=[
                pl.BlockSpec(memory_space=pl.ANY),
                pl.BlockSpec((1, _CTX * 128), lambda i, idx_ref: (0, 0)),
                pl.BlockSpec((_DIM, _BN), lambda i, idx_ref: (0, i)),
                pl.BlockSpec((1, _BN), lambda i, idx_ref: (0, i)),
            ],
            out_specs=[
                pl.BlockSpec((1, _BN), lambda i, idx_ref: (0, i)),
                pl.BlockSpec(memory_space=pltpu.SMEM),
            ],
            scratch_shapes=[
                pltpu.VMEM((_DIM, 1), jnp.float32),
                pltpu.VMEM((_DIM, _CTX * 128), jnp.float32),
                pltpu.SMEM((2,), jnp.float32),
                pltpu.SemaphoreType.DMA,
            ],
        ),
        out_shape=[
            jax.ShapeDtypeStruct((1, _VOCAB), jnp.float32),
            jax.ShapeDtypeStruct((1, 1), jnp.float32),
        ],
    )(idx, emb_table.T, mask, W.T, b.reshape(1, _VOCAB))

    out = pl.pallas_call(
        _sub_kernel,
        grid=(_NB,),
        in_specs=[
            pl.BlockSpec((1, _BN), lambda i: (0, i)),
            pl.BlockSpec(memory_space=pltpu.SMEM),
        ],
        out_specs=pl.BlockSpec((1, _BN), lambda i: (0, i)),
        out_shape=jax.ShapeDtypeStruct((1, _VOCAB), jnp.float32),
    )(logits, lse)

    return out


# speedup vs baseline: 12.9610x; 1.4689x over previous
import jax
import jax.numpy as jnp
from jax.experimental import pallas as pl
from jax.experimental.pallas import tpu as pltpu

_VOCAB = 1000000
_DIM = 64
_CTX = 200
_BN = 16384
_NB = -(-_VOCAB // _BN)


def _main_kernel(idx_ref, embt_hbm, mask_ref, wt_ref, b_ref, out_ref, lse_ref,
                 m_ref, cols_ref, stat_ref, sem):
    i = pl.program_id(0)

    @pl.when(i == 0)
    def _():
        def issue(j, c):
            base = pl.multiple_of(idx_ref[j] // 128 * 128, 128)
            pltpu.make_async_copy(
                embt_hbm.at[:, pl.ds(base, 128)],
                cols_ref.at[:, pl.ds(pl.multiple_of(128 * j, 128), 128)],
                sem).start()
            return c

        jax.lax.fori_loop(0, _CTX, issue, 0)

        def wait(j, c):
            base = pl.multiple_of(idx_ref[j] // 128 * 128, 128)
            pltpu.make_async_copy(
                embt_hbm.at[:, pl.ds(base, 128)],
                cols_ref.at[:, pl.ds(pl.multiple_of(128 * j, 128), 128)],
                sem).wait()
            return c

        jax.lax.fori_loop(0, _CTX, wait, 0)
        m_ref[...] = jnp.sum(cols_ref[...] * mask_ref[...], axis=1,
                             keepdims=True) * (1.0 / _CTX)
        stat_ref[0] = -jnp.inf
        stat_ref[1] = 0.0

    s = jax.lax.dot_general(
        m_ref[...], wt_ref[...], (((0,), (0,)), ((), ())),
        preferred_element_type=jnp.float32,
    ) + b_ref[...]
    out_ref[...] = s

    col = _BN * i + jax.lax.broadcasted_iota(jnp.int32, (1, _BN), 1)
    sm = jnp.where(col < _VOCAB, s, -jnp.inf)
    old_max = stat_ref[0]
    new_max = jnp.maximum(old_max, jnp.max(sm))
    stat_ref[1] = stat_ref[1] * jnp.exp(old_max - new_max) + jnp.sum(
        jnp.where(col < _VOCAB, jnp.exp(sm - new_max), 0.0))
    stat_ref[0] = new_max

    @pl.when(i == _NB - 1)
    def _():
        lse_ref[0, 0] = stat_ref[0] + jnp.log(stat_ref[1])


def _sub_kernel(x_ref, lse_ref, o_ref):
    o_ref[...] = x_ref[...] - lse_ref[0, 0]


@jax.jit
def kernel(inputs, emb_table, W, b):
    idx = inputs.astype(jnp.int32)
    onehot = (idx[:, None] % 128 ==
              jnp.arange(128, dtype=jnp.int32)[None, :]).astype(jnp.float32)
    mask = onehot.reshape(1, _CTX * 128)

    logits, lse = pl.pallas_call(
        _main_kernel,
        grid_spec=pltpu.PrefetchScalarGridSpec(
            num_scalar_prefetch=1,
            grid=(_NB,),
            in_specs=[
                pl.BlockSpec(memory_space=pl.ANY),
                pl.BlockSpec((1, _CTX * 128), lambda i, idx_ref: (0, 0)),
                pl.BlockSpec((_DIM, _BN), lambda i, idx_ref: (0, i)),
                pl.BlockSpec((1, _BN), lambda i, idx_ref: (0, i)),
            ],
            out_specs=[
                pl.BlockSpec((1, _BN), lambda i, idx_ref: (0, i)),
                pl.BlockSpec(memory_space=pltpu.SMEM),
            ],
            scratch_shapes=[
                pltpu.VMEM((_DIM, 1), jnp.float32),
                pltpu.VMEM((_DIM, _CTX * 128), jnp.float32),
                pltpu.SMEM((2,), jnp.float32),
                pltpu.SemaphoreType.DMA,
            ],
        ),
        out_shape=[
            jax.ShapeDtypeStruct((1, _VOCAB), jnp.float32),
            jax.ShapeDtypeStruct((1, 1), jnp.float32),
        ],
    )(idx, emb_table.T, mask, W.T, b.reshape(1, _VOCAB))

    out = pl.pallas_call(
        _sub_kernel,
        grid=(_NB,),
        in_specs=[
            pl.BlockSpec((1, _BN), lambda i: (0, i)),
            pl.BlockSpec(memory_space=pltpu.SMEM),
        ],
        out_specs=pl.BlockSpec((1, _BN), lambda i: (0, i)),
        out_shape=jax.ShapeDtypeStruct((1, _VOCAB), jnp.float32),
    )(logits, lse)

    return out


# speedup vs baseline: 16.7133x; 1.2895x over previous
import jax
import jax.numpy as jnp
from jax.experimental import pallas as pl
from jax.experimental.pallas import tpu as pltpu

_VOCAB = 1000000
_DIM = 64
_CTX = 200
_BN = 32768
_NB = -(-_VOCAB // _BN)


def _main_kernel(idx_ref, embt_hbm, mask_ref, wt_ref, b_ref, out_ref, lse_ref,
                 m_ref, cols_ref, stat_ref, sem):
    i = pl.program_id(0)

    @pl.when(i == 0)
    def _():
        def issue(j, c):
            base = pl.multiple_of(idx_ref[j] // 128 * 128, 128)
            pltpu.make_async_copy(
                embt_hbm.at[:, pl.ds(base, 128)],
                cols_ref.at[:, pl.ds(pl.multiple_of(128 * j, 128), 128)],
                sem).start()
            return c

        jax.lax.fori_loop(0, _CTX, issue, 0)

        def wait(j, c):
            base = pl.multiple_of(idx_ref[j] // 128 * 128, 128)
            pltpu.make_async_copy(
                embt_hbm.at[:, pl.ds(base, 128)],
                cols_ref.at[:, pl.ds(pl.multiple_of(128 * j, 128), 128)],
                sem).wait()
            return c

        jax.lax.fori_loop(0, _CTX, wait, 0)
        m_ref[...] = jnp.sum(cols_ref[...] * mask_ref[...], axis=1,
                             keepdims=True) * (1.0 / _CTX)
        stat_ref[0] = -jnp.inf
        stat_ref[1] = 0.0

    s = jax.lax.dot_general(
        m_ref[...], wt_ref[...], (((0,), (0,)), ((), ())),
        preferred_element_type=jnp.float32,
    ) + b_ref[...]
    out_ref[...] = s

    col = _BN * i + jax.lax.broadcasted_iota(jnp.int32, (1, _BN), 1)
    sm = jnp.where(col < _VOCAB, s, -jnp.inf)
    old_max = stat_ref[0]
    new_max = jnp.maximum(old_max, jnp.max(sm))
    stat_ref[1] = stat_ref[1] * jnp.exp(old_max - new_max) + jnp.sum(
        jnp.where(col < _VOCAB, jnp.exp(sm - new_max), 0.0))
    stat_ref[0] = new_max

    @pl.when(i == _NB - 1)
    def _():
        lse_ref[0, 0] = stat_ref[0] + jnp.log(stat_ref[1])


def _sub_kernel(x_ref, lse_ref, o_ref):
    o_ref[...] = x_ref[...] - lse_ref[0, 0]


@jax.jit
def kernel(inputs, emb_table, W, b):
    idx = inputs.astype(jnp.int32)
    onehot = (idx[:, None] % 128 ==
              jnp.arange(128, dtype=jnp.int32)[None, :]).astype(jnp.float32)
    mask = onehot.reshape(1, _CTX * 128)

    logits, lse = pl.pallas_call(
        _main_kernel,
        grid_spec=pltpu.PrefetchScalarGridSpec(
            num_scalar_prefetch=1,
            grid=(_NB,),
            in_specs=[
                pl.BlockSpec(memory_space=pl.ANY),
                pl.BlockSpec((1, _CTX * 128), lambda i, idx_ref: (0, 0)),
                pl.BlockSpec((_DIM, _BN), lambda i, idx_ref: (0, i)),
                pl.BlockSpec((1, _BN), lambda i, idx_ref: (0, i)),
            ],
            out_specs=[
                pl.BlockSpec((1, _BN), lambda i, idx_ref: (0, i)),
                pl.BlockSpec(memory_space=pltpu.SMEM),
            ],
            scratch_shapes=[
                pltpu.VMEM((_DIM, 1), jnp.float32),
                pltpu.VMEM((_DIM, _CTX * 128), jnp.float32),
                pltpu.SMEM((2,), jnp.float32),
                pltpu.SemaphoreType.DMA,
            ],
        ),
        out_shape=[
            jax.ShapeDtypeStruct((1, _VOCAB), jnp.float32),
            jax.ShapeDtypeStruct((1, 1), jnp.float32),
        ],
    )(idx, emb_table.T, mask, W.T, b.reshape(1, _VOCAB))

    out = pl.pallas_call(
        _sub_kernel,
        grid=(_NB,),
        in_specs=[
            pl.BlockSpec((1, _BN), lambda i: (0, i)),
            pl.BlockSpec(memory_space=pltpu.SMEM),
        ],
        out_specs=pl.BlockSpec((1, _BN), lambda i: (0, i)),
        out_shape=jax.ShapeDtypeStruct((1, _VOCAB), jnp.float32),
    )(logits, lse)

    return out


# speedup vs baseline: 17.2863x; 1.0343x over previous
import jax
import jax.numpy as jnp
from jax.experimental import pallas as pl
from jax.experimental.pallas import tpu as pltpu

_VOCAB = 1000000
_DIM = 64
_CTX = 200
_BN = 65536
_NB = -(-_VOCAB // _BN)


def _main_kernel(idx_ref, embt_hbm, mask_ref, wt_ref, b_ref, out_ref, lse_ref,
                 m_ref, cols_ref, stat_ref, sem):
    i = pl.program_id(0)

    @pl.when(i == 0)
    def _():
        def issue(j, c):
            base = pl.multiple_of(idx_ref[j] // 128 * 128, 128)
            pltpu.make_async_copy(
                embt_hbm.at[:, pl.ds(base, 128)],
                cols_ref.at[:, pl.ds(pl.multiple_of(128 * j, 128), 128)],
                sem).start()
            return c

        jax.lax.fori_loop(0, _CTX, issue, 0)

        def wait(j, c):
            base = pl.multiple_of(idx_ref[j] // 128 * 128, 128)
            pltpu.make_async_copy(
                embt_hbm.at[:, pl.ds(base, 128)],
                cols_ref.at[:, pl.ds(pl.multiple_of(128 * j, 128), 128)],
                sem).wait()
            return c

        jax.lax.fori_loop(0, _CTX, wait, 0)
        m_ref[...] = jnp.sum(cols_ref[...] * mask_ref[...], axis=1,
                             keepdims=True) * (1.0 / _CTX)
        stat_ref[0] = -jnp.inf
        stat_ref[1] = 0.0

    s = jax.lax.dot_general(
        m_ref[...], wt_ref[...], (((0,), (0,)), ((), ())),
        preferred_element_type=jnp.float32,
    ) + b_ref[...]
    out_ref[...] = s

    col = _BN * i + jax.lax.broadcasted_iota(jnp.int32, (1, _BN), 1)
    sm = jnp.where(col < _VOCAB, s, -jnp.inf)
    old_max = stat_ref[0]
    new_max = jnp.maximum(old_max, jnp.max(sm))
    stat_ref[1] = stat_ref[1] * jnp.exp(old_max - new_max) + jnp.sum(
        jnp.where(col < _VOCAB, jnp.exp(sm - new_max), 0.0))
    stat_ref[0] = new_max

    @pl.when(i == _NB - 1)
    def _():
        lse_ref[0, 0] = stat_ref[0] + jnp.log(stat_ref[1])


def _sub_kernel(x_ref, lse_ref, o_ref):
    o_ref[...] = x_ref[...] - lse_ref[0, 0]


@jax.jit
def kernel(inputs, emb_table, W, b):
    idx = inputs.astype(jnp.int32)
    onehot = (idx[:, None] % 128 ==
              jnp.arange(128, dtype=jnp.int32)[None, :]).astype(jnp.float32)
    mask = onehot.reshape(1, _CTX * 128)

    logits, lse = pl.pallas_call(
        _main_kernel,
        grid_spec=pltpu.PrefetchScalarGridSpec(
            num_scalar_prefetch=1,
            grid=(_NB,),
            in_specs=[
                pl.BlockSpec(memory_space=pl.ANY),
                pl.BlockSpec((1, _CTX * 128), lambda i, idx_ref: (0, 0)),
                pl.BlockSpec((_DIM, _BN), lambda i, idx_ref: (0, i)),
                pl.BlockSpec((1, _BN), lambda i, idx_ref: (0, i)),
            ],
            out_specs=[
                pl.BlockSpec((1, _BN), lambda i, idx_ref: (0, i)),
                pl.BlockSpec(memory_space=pltpu.SMEM),
            ],
            scratch_shapes=[
                pltpu.VMEM((_DIM, 1), jnp.float32),
                pltpu.VMEM((_DIM, _CTX * 128), jnp.float32),
                pltpu.SMEM((2,), jnp.float32),
                pltpu.SemaphoreType.DMA,
            ],
        ),
        out_shape=[
            jax.ShapeDtypeStruct((1, _VOCAB), jnp.float32),
            jax.ShapeDtypeStruct((1, 1), jnp.float32),
        ],
    )(idx, emb_table.T, mask, W.T, b.reshape(1, _VOCAB))

    out = pl.pallas_call(
        _sub_kernel,
        grid=(_NB,),
        in_specs=[
            pl.BlockSpec((1, _BN), lambda i: (0, i)),
            pl.BlockSpec(memory_space=pltpu.SMEM),
        ],
        out_specs=pl.BlockSpec((1, _BN), lambda i: (0, i)),
        out_shape=jax.ShapeDtypeStruct((1, _VOCAB), jnp.float32),
    )(logits, lse)

    return out
